# baseline (speedup 1.0000x reference)
import jax
import jax.numpy as jnp
import numpy as np
from jax import lax
from jax.experimental import pallas as pl
from jax.experimental.pallas import tpu as pltpu
from jax.experimental.pallas import tpu_sc as plsc

N = 10000
E = 320000
C = 128
OUT = 128
SEEDS = 1024

NC = 2
NS = 16
NW = NC * NS
L = 16

EW = E // NW
EK = 80
ENCHUNK = EW // EK
ZSTRIPE = 640
ZTAIL = N - (NS - 1) * ZSTRIPE

EK2 = 112
CAP2 = 10112
TRASH = SEEDS
AGG2_ROWS = 1152
Z2STRIPE = AGG2_ROWS // NS


def _sc_mesh():
    return plsc.VectorSubcoreMesh(core_axis_name="c", subcore_axis_name="s",
                                  num_cores=NC, num_subcores=NS)


def _sc_scatter(h, src, dst, zeros_hbm):
    def body(h_hbm, src_hbm, dst_hbm, zeros_hbm,
             out_hbm, cpk_hbm, cnt_hbm,
             sidx0, sidx1, sidx2, didx0, didx1, didx2,
             rows0, rows1, rows2, cbuf, cnt_v, agg,
             gsem0, gsem1, gsem2, ssem0, ssem1, ssem2,
             isem0, isem1, isem2, dsem0, dsem1, dsem2):
        sidx = [sidx0, sidx1, sidx2]
        didx = [didx0, didx1, didx2]
        rows = [rows0, rows1, rows2]
        gsem = [gsem0, gsem1, gsem2]
        ssem = [ssem0, ssem1, ssem2]
        isem = [isem0, isem1, isem2]
        dsem = [dsem0, dsem1, dsem2]
        c = lax.axis_index("c")
        s = lax.axis_index("s")
        wid = s * NC + c

        @pl.when(s < NS - 1)
        def _():
            pltpu.sync_copy(zeros_hbm, agg.at[pl.ds(s * ZSTRIPE, ZSTRIPE)])

        @pl.when(s == NS - 1)
        def _():
            pltpu.sync_copy(zeros_hbm.at[pl.ds(0, ZTAIL)],
                            agg.at[pl.ds((NS - 1) * ZSTRIPE, ZTAIL)])

        plsc.subcore_barrier()

        e0 = wid * EW
        for kp in range(3):
            pltpu.sync_copy(src_hbm.at[pl.ds(e0 + kp * EK, EK)], sidx[kp])
        for kp in range(2):
            pltpu.sync_copy(dst_hbm.at[pl.ds(e0 + kp * EK, EK)], didx[kp])

        def fire_sidx(k, b):
            pltpu.async_copy(src_hbm.at[pl.ds(e0 + k * EK, EK)],
                             sidx[b], isem[b])

        def wait_sidx(b):
            pltpu.make_async_copy(src_hbm.at[pl.ds(0, EK)], sidx[b],
                                  isem[b]).wait()

        def fire_didx(k, b):
            pltpu.async_copy(dst_hbm.at[pl.ds(e0 + k * EK, EK)],
                             didx[b], dsem[b])

        def wait_didx(b):
            pltpu.make_async_copy(dst_hbm.at[pl.ds(0, EK)], didx[b],
                                  dsem[b]).wait()

        def fire_gather(b):
            pltpu.async_copy(h_hbm.at[sidx[b]], rows[b], gsem[b])

        def wait_gather(b):
            pltpu.make_async_copy(h_hbm.at[pl.ds(0, EK)], rows[b],
                                  gsem[b]).wait()

        def drain_scatter(b):
            pltpu.make_async_copy(h_hbm.at[pl.ds(0, EK)], rows[b],
                                  ssem[b]).wait()

        def compact(b, off_v):
            for i in range(EK // L):
                sv = sidx[b][pl.ds(i * L, L)]
                dv = didx[b][pl.ds(i * L, L)]
                m = dv < SEEDS
                cum = plsc.cumsum(m.astype(jnp.int32))
                pos = off_v + cum - 1
                plsc.store_scatter(cbuf, [pos], sv | (dv << 16), mask=m)
                off_v = off_v + plsc.all_reduce_population_count(m)
            return off_v

        fire_gather(0)
        fire_gather(1)

        def triple(j, off_v):
            for b in range(3):
                k = 3 * j + b
                wait_gather(b)

                @pl.when(k >= 2)
                def _():
                    wait_didx(b)

                pltpu.async_copy(rows[b], agg.at[didx[b]], ssem[b],
                                 add=True)
                off_v = compact(b, off_v)

                @pl.when(k + 3 < ENCHUNK)
                def _():
                    fire_sidx(k + 3, b)

                @pl.when(k >= 1)
                def _():
                    drain_scatter((b + 2) % 3)

                @pl.when(k + 2 < ENCHUNK)
                def _():
                    fire_didx(k + 2, (b + 2) % 3)

                @pl.when(k >= 1)
                def _():
                    wait_sidx((b + 2) % 3)

                fire_gather((b + 2) % 3)
            return off_v

        nmain = (ENCHUNK - 2) // 3
        off_v = lax.fori_loop(0, nmain, triple, jnp.zeros((L,), jnp.int32))
        for k in range(3 * nmain, ENCHUNK):
            b = k % 3
            wait_gather(b)
            wait_didx(b)
            pltpu.async_copy(rows[b], agg.at[didx[b]], ssem[b], add=True)
            off_v = compact(b, off_v)
            drain_scatter((b + 2) % 3)
        drain_scatter((ENCHUNK - 1) % 3)

        ramp = lax.iota(jnp.int32, L)
        trash_v = jnp.zeros((L,), jnp.int32) + (TRASH << 16)
        for i in range(EK2 // L):
            plsc.store_scatter(cbuf, [off_v + i * L + ramp], trash_v)
        off = jnp.max(off_v)
        cnt_v[...] = jnp.zeros((L,), jnp.int32) + (off + EK2 - 1) // EK2
        pltpu.sync_copy(cbuf, cpk_hbm.at[wid])
        pltpu.sync_copy(cnt_v, cnt_hbm.at[wid])

        plsc.subcore_barrier()
        @pl.when(s < NS - 1)
        def _():
            pltpu.sync_copy(agg.at[pl.ds(s * ZSTRIPE, ZSTRIPE)],
                            out_hbm.at[c, pl.ds(s * ZSTRIPE, ZSTRIPE)])

        @pl.when(s == NS - 1)
        def _():
            pltpu.sync_copy(agg.at[pl.ds((NS - 1) * ZSTRIPE, ZTAIL)],
                            out_hbm.at[c, pl.ds((NS - 1) * ZSTRIPE, ZTAIL)])

    k = pl.kernel(
        body,
        out_type=(
            jax.ShapeDtypeStruct((NC, N, C), jnp.float32),
            jax.ShapeDtypeStruct((NW, CAP2), jnp.int32),
            jax.ShapeDtypeStruct((NW, L), jnp.int32),
        ),
        mesh=_sc_mesh(),
        scratch_types=(
            [pltpu.VMEM((EK,), jnp.int32) for _ in range(6)]
            + [pltpu.VMEM((EK, C), jnp.float32) for _ in range(3)]
            + [pltpu.VMEM((CAP2,), jnp.int32),
               pltpu.VMEM((L,), jnp.int32)]
            + [pltpu.VMEM_SHARED((N, C), jnp.float32)]
            + [pltpu.SemaphoreType.DMA for _ in range(12)]
        ),
        compiler_params=pltpu.CompilerParams(needs_layout_passes=False),
    )
    return k(h, src, dst, zeros_hbm)


def _sc_scatter_seeds(h1, cpk, cnt, zeros_hbm):
    def body(h_hbm, cpk_hbm, cnt_hbm, zeros_hbm, out_hbm,
             sv, cnt_v, si_all, di2d,
             rows0, rows1, rows2, rows3, rows4, rows5, agg,
             gsem0, gsem1, gsem2, gsem3, gsem4, gsem5,
             ssem0, ssem1, ssem2, ssem3, ssem4, ssem5):
        rows = [rows0, rows1, rows2, rows3, rows4, rows5]
        gsem = [gsem0, gsem1, gsem2, gsem3, gsem4, gsem5]
        ssem = [ssem0, ssem1, ssem2, ssem3, ssem4, ssem5]
        c = lax.axis_index("c")
        s = lax.axis_index("s")
        wid = s * NC + c

        pltpu.sync_copy(zeros_hbm.at[pl.ds(0, Z2STRIPE)],
                        agg.at[pl.ds(s * Z2STRIPE, Z2STRIPE)])
        plsc.subcore_barrier()

        pltpu.sync_copy(cpk_hbm.at[wid], sv)
        pltpu.sync_copy(cnt_hbm.at[wid], cnt_v)
        nch = jnp.max(cnt_v[...])

        ramp = lax.iota(jnp.int32, L)
        krow = jnp.zeros((L,), jnp.int32)

        def unpack(k, _):
            for i in range(EK2 // L):
                pv = sv[pl.ds(k * EK2 + i * L, L)]
                plsc.store_scatter(si_all, [k * EK2 + i * L + ramp],
                                   pv & 0xFFFF)
                plsc.store_scatter(di2d, [krow + k, i * L + ramp],
                                   pv >> 16)
            return _

        lax.fori_loop(0, nch, unpack, None)

        def fire_gather(k, b):
            pltpu.async_copy(h_hbm.at[si_all.at[pl.ds(k * EK2, EK2)]],
                             rows[b], gsem[b])

        def wait_gather(b):
            pltpu.make_async_copy(h_hbm.at[pl.ds(0, EK2)], rows[b],
                                  gsem[b]).wait()

        def drain_scatter(b):
            pltpu.make_async_copy(h_hbm.at[pl.ds(0, EK2)], rows[b],
                                  ssem[b]).wait()

        for kp in range(4):
            @pl.when(kp < nch)
            def _():
                fire_gather(kp, kp)

        def hexa(j, _):
            for b in range(6):
                k = 6 * j + b

                @pl.when(k < nch)
                def _():
                    wait_gather(b)
                    pltpu.async_copy(rows[b], agg.at[di2d.at[k]], ssem[b],
                                     add=True)

                    @pl.when(k >= 2)
                    def _():
                        drain_scatter((b + 4) % 6)

                    @pl.when(k + 4 < nch)
                    def _():
                        fire_gather(k + 4, (b + 4) % 6)
            return _

        lax.fori_loop(0, (nch + 5) // 6, hexa, None)
        for d in range(6):
            @pl.when((nch >= 1) & ((nch - 1) % 6 == d))
            def _():
                drain_scatter(d)
        for d in range(6):
            @pl.when((nch >= 2) & ((nch - 2) % 6 == d))
            def _():
                drain_scatter(d)

        plsc.subcore_barrier()
        wpt = SEEDS // NS
        pltpu.sync_copy(agg.at[pl.ds(s * wpt, wpt)],
                        out_hbm.at[c, pl.ds(s * wpt, wpt)])

    k = pl.kernel(
        body,
        out_type=jax.ShapeDtypeStruct((NC, SEEDS, C), jnp.float32),
        mesh=_sc_mesh(),
        scratch_types=(
            [pltpu.VMEM((CAP2,), jnp.int32),
             pltpu.VMEM((L,), jnp.int32),
             pltpu.VMEM((CAP2,), jnp.int32),
             pltpu.VMEM((CAP2 // EK2 + 1, EK2), jnp.int32)]
            + [pltpu.VMEM((EK2, C), jnp.float32) for _ in range(6)]
            + [pltpu.VMEM_SHARED((AGG2_ROWS, C), jnp.float32)]
            + [pltpu.SemaphoreType.DMA for _ in range(12)]
        ),
        compiler_params=pltpu.CompilerParams(needs_layout_passes=False),
    )
    return k(h1, cpk, cnt, zeros_hbm)


_RBLK = 2000


def _encoder_body(x_ref, bid_ref, nt_ref, seed_ref,
                  wf_ref, bf_ref, wn_ref, bn_ref,
                  lg_ref, lb_ref, wt_ref, bt_ref, o_ref):
    x = x_ref[...]
    h = jnp.dot(x, wf_ref[...], preferred_element_type=jnp.float32) + bf_ref[...]
    t = jnp.dot(h, wn_ref[...], preferred_element_type=jnp.float32) + bn_ref[...]
    t = jnp.maximum(t, 0.0)
    mu = jnp.mean(t, axis=-1, keepdims=True)
    var = jnp.mean((t - mu) ** 2, axis=-1, keepdims=True)
    t = (t - mu) * lax.rsqrt(var + 1e-5) * lg_ref[...] + lb_ref[...]
    iot = lax.broadcasted_iota(jnp.int32, (_RBLK, SEEDS), 1)
    oh = jnp.where(iot == bid_ref[...], seed_ref[...], 0.0)
    rel = jnp.sum(oh, axis=1, keepdims=True) - nt_ref[...]
    half = C // 2
    f = lax.broadcasted_iota(jnp.int32, (1, half), 1).astype(jnp.float32)
    freqs = jnp.exp(f * (-np.log(10000.0) / half))
    ang = rel * freqs
    pe = jnp.concatenate([jnp.sin(ang), jnp.cos(ang)], axis=-1)
    o_ref[...] = t + jnp.dot(pe, wt_ref[...],
                             preferred_element_type=jnp.float32) + bt_ref[...]


def _tc_encoder(x, batch_ids, node_time, seed_time,
                W_feat, b_feat, W_node, b_node, ln_g, ln_b,
                W_time, b_time):
    grid = N // _RBLK
    w2 = pl.BlockSpec((C, C), lambda i: (0, 0))
    w1 = pl.BlockSpec((C,), lambda i: (0,))
    return pl.pallas_call(
        _encoder_body,
        grid=(grid,),
        in_specs=[
            pl.BlockSpec((_RBLK, C), lambda i: (i, 0)),
            pl.BlockSpec((_RBLK, 1), lambda i: (i, 0)),
            pl.BlockSpec((_RBLK, 1), lambda i: (i, 0)),
            pl.BlockSpec((1, SEEDS), lambda i: (0, 0)),
            w2, w1, w2, w1, w1, w1, w2, w1,
        ],
        out_specs=pl.BlockSpec((_RBLK, C), lambda i: (i, 0)),
        out_shape=jax.ShapeDtypeStruct((N, C), jnp.float32),
    )(x, batch_ids, node_time, seed_time,
      W_feat, b_feat, W_node, b_node, ln_g, ln_b, W_time, b_time)


def _layer_body(h_ref, p0_ref, p1_ref, ws_ref, wn_ref, b_ref, o_ref):
    agg = p0_ref[0] + p1_ref[0]
    o = (jnp.dot(h_ref[...], ws_ref[...], preferred_element_type=jnp.float32)
         + jnp.dot(agg, wn_ref[...], preferred_element_type=jnp.float32)
         + b_ref[...])
    o_ref[...] = jnp.maximum(o, 0.0)


def _tc_layer1(h, p, W_self, W_neigh, b):
    grid = N // _RBLK
    blk = pl.BlockSpec((_RBLK, C), lambda i: (i, 0))
    p0s = pl.BlockSpec((1, _RBLK, C), lambda i: (0, i, 0))
    p1s = pl.BlockSpec((1, _RBLK, C), lambda i: (1, i, 0))
    w2 = pl.BlockSpec((C, C), lambda i: (0, 0))
    w1 = pl.BlockSpec((C,), lambda i: (0,))
    return pl.pallas_call(
        _layer_body,
        grid=(grid,),
        in_specs=[blk, p0s, p1s, w2, w2, w1],
        out_specs=blk,
        out_shape=jax.ShapeDtypeStruct((N, C), jnp.float32),
    )(h, p, p, W_self, W_neigh, b)


def _head_body(h_ref, q0_ref, q1_ref, ws_ref, wn_ref, b_ref,
               wh_ref, bh_ref, o_ref):
    agg = q0_ref[0] + q1_ref[0]
    t = (jnp.dot(h_ref[...], ws_ref[...], preferred_element_type=jnp.float32)
         + jnp.dot(agg, wn_ref[...], preferred_element_type=jnp.float32)
         + b_ref[...])
    t = jnp.maximum(t, 0.0)
    o_ref[...] = jnp.dot(t, wh_ref[...],
                         preferred_element_type=jnp.float32) + bh_ref[...]


def _tc_head(h1, q, W_self, W_neigh, b, W_head, b_head):
    return pl.pallas_call(
        _head_body,
        grid=(1,),
        in_specs=[
            pl.BlockSpec((SEEDS, C), lambda i: (0, 0)),
            pl.BlockSpec((1, SEEDS, C), lambda i: (0, 0, 0)),
            pl.BlockSpec((1, SEEDS, C), lambda i: (1, 0, 0)),
            pl.BlockSpec((C, C), lambda i: (0, 0)),
            pl.BlockSpec((C, C), lambda i: (0, 0)),
            pl.BlockSpec((C,), lambda i: (0,)),
            pl.BlockSpec((C, OUT), lambda i: (0, 0)),
            pl.BlockSpec((OUT,), lambda i: (0,)),
        ],
        out_specs=pl.BlockSpec((SEEDS, OUT), lambda i: (0, 0)),
        out_shape=jax.ShapeDtypeStruct((SEEDS, OUT), jnp.float32),
    )(h1, q, q, W_self, W_neigh, b, W_head, b_head)


def kernel(x, edge_index, node_time, seed_time, batch_ids,
           W_feat, b_feat, W_node, b_node, ln_g, ln_b,
           W_time, b_time,
           W_self1, W_neigh1, b1, W_self2, W_neigh2, b2,
           W_head, b_head):
    h = _tc_encoder(x, batch_ids.reshape(N, 1), node_time.reshape(N, 1),
                    seed_time.reshape(1, SEEDS),
                    W_feat, b_feat, W_node, b_node, ln_g, ln_b,
                    W_time, b_time)

    zeros_hbm = jnp.zeros((ZSTRIPE, C), jnp.float32)
    src = edge_index[0]
    dst = edge_index[1]
    p, cpk, cnt = _sc_scatter(h, src, dst, zeros_hbm)
    h1 = _tc_layer1(h, p, W_self1, W_neigh1, b1)

    q = _sc_scatter_seeds(h1, cpk, cnt, zeros_hbm)
    return _tc_head(h1, q, W_self2, W_neigh2, b2, W_head, b_head)


# speedup vs baseline: 1.1240x; 1.1240x over previous
import jax
import jax.numpy as jnp
import numpy as np
from jax import lax
from jax.experimental import pallas as pl
from jax.experimental.pallas import tpu as pltpu
from jax.experimental.pallas import tpu_sc as plsc

N = 10000
E = 320000
C = 128
OUT = 128
SEEDS = 1024

NC = 2
NS = 16
NW = NC * NS
L = 16

EW = E // NW
EK = 80
ENCHUNK = EW // EK
ZSTRIPE = 640
ZTAIL = N - (NS - 1) * ZSTRIPE

EK2 = 80
CAP2 = 10080
TRASH = SEEDS
AGG2_ROWS = 1152
Z2STRIPE = AGG2_ROWS // NS


def _sc_mesh():
    return plsc.VectorSubcoreMesh(core_axis_name="c", subcore_axis_name="s",
                                  num_cores=NC, num_subcores=NS)


def _sc_scatter(h, src, dst, zeros_hbm):
    def body(h_hbm, src_hbm, dst_hbm, zeros_hbm,
             out_hbm, cpk_hbm, cnt_hbm,
             sidx0, sidx1, sidx2, didx0, didx1, didx2,
             rows0, rows1, rows2, cbuf, cnt_v, agg,
             gsem0, gsem1, gsem2, ssem0, ssem1, ssem2,
             isem0, isem1, isem2, dsem0, dsem1, dsem2):
        sidx = [sidx0, sidx1, sidx2]
        didx = [didx0, didx1, didx2]
        rows = [rows0, rows1, rows2]
        gsem = [gsem0, gsem1, gsem2]
        ssem = [ssem0, ssem1, ssem2]
        isem = [isem0, isem1, isem2]
        dsem = [dsem0, dsem1, dsem2]
        c = lax.axis_index("c")
        s = lax.axis_index("s")
        wid = s * NC + c

        @pl.when(s < NS - 1)
        def _():
            pltpu.sync_copy(zeros_hbm, agg.at[pl.ds(s * ZSTRIPE, ZSTRIPE)])

        @pl.when(s == NS - 1)
        def _():
            pltpu.sync_copy(zeros_hbm.at[pl.ds(0, ZTAIL)],
                            agg.at[pl.ds((NS - 1) * ZSTRIPE, ZTAIL)])

        plsc.subcore_barrier()

        e0 = wid * EW
        for kp in range(3):
            pltpu.sync_copy(src_hbm.at[pl.ds(e0 + kp * EK, EK)], sidx[kp])
        for kp in range(2):
            pltpu.sync_copy(dst_hbm.at[pl.ds(e0 + kp * EK, EK)], didx[kp])

        def fire_sidx(k, b):
            pltpu.async_copy(src_hbm.at[pl.ds(e0 + k * EK, EK)],
                             sidx[b], isem[b])

        def wait_sidx(b):
            pltpu.make_async_copy(src_hbm.at[pl.ds(0, EK)], sidx[b],
                                  isem[b]).wait()

        def fire_didx(k, b):
            pltpu.async_copy(dst_hbm.at[pl.ds(e0 + k * EK, EK)],
                             didx[b], dsem[b])

        def wait_didx(b):
            pltpu.make_async_copy(dst_hbm.at[pl.ds(0, EK)], didx[b],
                                  dsem[b]).wait()

        def fire_gather(b):
            pltpu.async_copy(h_hbm.at[sidx[b]], rows[b], gsem[b])

        def wait_gather(b):
            pltpu.make_async_copy(h_hbm.at[pl.ds(0, EK)], rows[b],
                                  gsem[b]).wait()

        def drain_scatter(b):
            pltpu.make_async_copy(h_hbm.at[pl.ds(0, EK)], rows[b],
                                  ssem[b]).wait()

        def compact(b, off_v):
            for i in range(EK // L):
                sv = sidx[b][pl.ds(i * L, L)]
                dv = didx[b][pl.ds(i * L, L)]
                m = dv < SEEDS
                cum = plsc.cumsum(m.astype(jnp.int32))
                pos = off_v + cum - 1
                plsc.store_scatter(cbuf, [pos], sv | (dv << 16), mask=m)
                off_v = off_v + plsc.all_reduce_population_count(m)
            return off_v

        fire_gather(0)
        fire_gather(1)

        def triple(j, off_v):
            for b in range(3):
                k = 3 * j + b
                wait_gather(b)

                @pl.when(k >= 2)
                def _():
                    wait_didx(b)

                pltpu.async_copy(rows[b], agg.at[didx[b]], ssem[b],
                                 add=True)
                off_v = compact(b, off_v)

                @pl.when(k + 3 < ENCHUNK)
                def _():
                    fire_sidx(k + 3, b)

                @pl.when(k >= 1)
                def _():
                    drain_scatter((b + 2) % 3)

                @pl.when(k + 2 < ENCHUNK)
                def _():
                    fire_didx(k + 2, (b + 2) % 3)

                @pl.when(k >= 1)
                def _():
                    wait_sidx((b + 2) % 3)

                fire_gather((b + 2) % 3)
            return off_v

        nmain = (ENCHUNK - 2) // 3
        off_v = lax.fori_loop(0, nmain, triple, jnp.zeros((L,), jnp.int32))
        for k in range(3 * nmain, ENCHUNK):
            b = k % 3
            wait_gather(b)
            wait_didx(b)
            pltpu.async_copy(rows[b], agg.at[didx[b]], ssem[b], add=True)
            off_v = compact(b, off_v)
            drain_scatter((b + 2) % 3)
        drain_scatter((ENCHUNK - 1) % 3)

        ramp = lax.iota(jnp.int32, L)
        trash_v = jnp.zeros((L,), jnp.int32) + (TRASH << 16)
        for i in range(EK2 // L):
            plsc.store_scatter(cbuf, [off_v + i * L + ramp], trash_v)
        off = jnp.max(off_v)
        cnt_v[...] = jnp.zeros((L,), jnp.int32) + (off + EK2 - 1) // EK2
        pltpu.sync_copy(cbuf, cpk_hbm.at[wid])
        pltpu.sync_copy(cnt_v, cnt_hbm.at[wid])

        plsc.subcore_barrier()
        @pl.when(s < NS - 1)
        def _():
            pltpu.sync_copy(agg.at[pl.ds(s * ZSTRIPE, ZSTRIPE)],
                            out_hbm.at[c, pl.ds(s * ZSTRIPE, ZSTRIPE)])

        @pl.when(s == NS - 1)
        def _():
            pltpu.sync_copy(agg.at[pl.ds((NS - 1) * ZSTRIPE, ZTAIL)],
                            out_hbm.at[c, pl.ds((NS - 1) * ZSTRIPE, ZTAIL)])

    k = pl.kernel(
        body,
        out_type=(
            jax.ShapeDtypeStruct((NC, N, C), jnp.float32),
            jax.ShapeDtypeStruct((NW, CAP2), jnp.int32),
            jax.ShapeDtypeStruct((NW, L), jnp.int32),
        ),
        mesh=_sc_mesh(),
        scratch_types=(
            [pltpu.VMEM((EK,), jnp.int32) for _ in range(6)]
            + [pltpu.VMEM((EK, C), jnp.float32) for _ in range(3)]
            + [pltpu.VMEM((CAP2,), jnp.int32),
               pltpu.VMEM((L,), jnp.int32)]
            + [pltpu.VMEM_SHARED((N, C), jnp.float32)]
            + [pltpu.SemaphoreType.DMA for _ in range(12)]
        ),
        compiler_params=pltpu.CompilerParams(needs_layout_passes=False),
    )
    return k(h, src, dst, zeros_hbm)


def _sc_scatter_seeds(h1, cpk, cnt, zeros_hbm):
    def body(h_hbm, cpk_hbm, cnt_hbm, zeros_hbm, out_hbm,
             sv, cnt_v, si_all, di2d,
             rows0, rows1, rows2, rows3, rows4, rows5, agg,
             gsem0, gsem1, gsem2, gsem3, gsem4, gsem5,
             ssem0, ssem1, ssem2, ssem3, ssem4, ssem5):
        rows = [rows0, rows1, rows2, rows3, rows4, rows5]
        gsem = [gsem0, gsem1, gsem2, gsem3, gsem4, gsem5]
        ssem = [ssem0, ssem1, ssem2, ssem3, ssem4, ssem5]
        c = lax.axis_index("c")
        s = lax.axis_index("s")
        wid = s * NC + c

        pltpu.sync_copy(zeros_hbm.at[pl.ds(0, Z2STRIPE)],
                        agg.at[pl.ds(s * Z2STRIPE, Z2STRIPE)])
        plsc.subcore_barrier()

        pltpu.sync_copy(cpk_hbm.at[wid], sv)
        pltpu.sync_copy(cnt_hbm.at[wid], cnt_v)
        nch = jnp.max(cnt_v[...])

        ramp = lax.iota(jnp.int32, L)
        krow = jnp.zeros((L,), jnp.int32)

        def unpack(k, _):
            for i in range(EK2 // L):
                pv = sv[pl.ds(k * EK2 + i * L, L)]
                plsc.store_scatter(si_all, [k * EK2 + i * L + ramp],
                                   pv & 0xFFFF)
                plsc.store_scatter(di2d, [krow + k, i * L + ramp],
                                   pv >> 16)
            return _

        lax.fori_loop(0, nch, unpack, None)

        def fire_gather(k, b):
            pltpu.async_copy(h_hbm.at[si_all.at[pl.ds(k * EK2, EK2)]],
                             rows[b], gsem[b])

        def wait_gather(b):
            pltpu.make_async_copy(h_hbm.at[pl.ds(0, EK2)], rows[b],
                                  gsem[b]).wait()

        def drain_scatter(b):
            pltpu.make_async_copy(h_hbm.at[pl.ds(0, EK2)], rows[b],
                                  ssem[b]).wait()

        for kp in range(4):
            @pl.when(kp < nch)
            def _():
                fire_gather(kp, kp)

        def hexa(j, _):
            for b in range(6):
                k = 6 * j + b

                @pl.when(k < nch)
                def _():
                    wait_gather(b)
                    pltpu.async_copy(rows[b], agg.at[di2d.at[k]], ssem[b],
                                     add=True)

                    @pl.when(k >= 2)
                    def _():
                        drain_scatter((b + 4) % 6)

                    @pl.when(k + 4 < nch)
                    def _():
                        fire_gather(k + 4, (b + 4) % 6)
            return _

        lax.fori_loop(0, (nch + 5) // 6, hexa, None)
        for d in range(6):
            @pl.when((nch >= 1) & ((nch - 1) % 6 == d))
            def _():
                drain_scatter(d)
        for d in range(6):
            @pl.when((nch >= 2) & ((nch - 2) % 6 == d))
            def _():
                drain_scatter(d)

        plsc.subcore_barrier()
        wpt = SEEDS // NS
        pltpu.sync_copy(agg.at[pl.ds(s * wpt, wpt)],
                        out_hbm.at[c, pl.ds(s * wpt, wpt)])

    k = pl.kernel(
        body,
        out_type=jax.ShapeDtypeStruct((NC, SEEDS, C), jnp.float32),
        mesh=_sc_mesh(),
        scratch_types=(
            [pltpu.VMEM((CAP2,), jnp.int32),
             pltpu.VMEM((L,), jnp.int32),
             pltpu.VMEM((CAP2,), jnp.int32),
             pltpu.VMEM((CAP2 // EK2 + 1, EK2), jnp.int32)]
            + [pltpu.VMEM((EK2, C), jnp.float32) for _ in range(6)]
            + [pltpu.VMEM_SHARED((AGG2_ROWS, C), jnp.float32)]
            + [pltpu.SemaphoreType.DMA for _ in range(12)]
        ),
        compiler_params=pltpu.CompilerParams(needs_layout_passes=False),
    )
    return k(h1, cpk, cnt, zeros_hbm)


_RBLK = 2000


def _encoder_body(x_ref, bid_ref, nt_ref, seed_ref,
                  wf_ref, bf_ref, wn_ref, bn_ref,
                  lg_ref, lb_ref, wt_ref, bt_ref, o_ref):
    x = x_ref[...]
    h = jnp.dot(x, wf_ref[...], preferred_element_type=jnp.float32) + bf_ref[...]
    t = jnp.dot(h, wn_ref[...], preferred_element_type=jnp.float32) + bn_ref[...]
    t = jnp.maximum(t, 0.0)
    mu = jnp.mean(t, axis=-1, keepdims=True)
    var = jnp.mean((t - mu) ** 2, axis=-1, keepdims=True)
    t = (t - mu) * lax.rsqrt(var + 1e-5) * lg_ref[...] + lb_ref[...]
    iot = lax.broadcasted_iota(jnp.int32, (_RBLK, SEEDS), 1)
    oh = jnp.where(iot == bid_ref[...], seed_ref[...], 0.0)
    rel = jnp.sum(oh, axis=1, keepdims=True) - nt_ref[...]
    half = C // 2
    f = lax.broadcasted_iota(jnp.int32, (1, half), 1).astype(jnp.float32)
    freqs = jnp.exp(f * (-np.log(10000.0) / half))
    ang = rel * freqs
    pe = jnp.concatenate([jnp.sin(ang), jnp.cos(ang)], axis=-1)
    o_ref[...] = t + jnp.dot(pe, wt_ref[...],
                             preferred_element_type=jnp.float32) + bt_ref[...]


def _tc_encoder(x, batch_ids, node_time, seed_time,
                W_feat, b_feat, W_node, b_node, ln_g, ln_b,
                W_time, b_time):
    grid = N // _RBLK
    w2 = pl.BlockSpec((C, C), lambda i: (0, 0))
    w1 = pl.BlockSpec((C,), lambda i: (0,))
    return pl.pallas_call(
        _encoder_body,
        grid=(grid,),
        in_specs=[
            pl.BlockSpec((_RBLK, C), lambda i: (i, 0)),
            pl.BlockSpec((_RBLK, 1), lambda i: (i, 0)),
            pl.BlockSpec((_RBLK, 1), lambda i: (i, 0)),
            pl.BlockSpec((1, SEEDS), lambda i: (0, 0)),
            w2, w1, w2, w1, w1, w1, w2, w1,
        ],
        out_specs=pl.BlockSpec((_RBLK, C), lambda i: (i, 0)),
        out_shape=jax.ShapeDtypeStruct((N, C), jnp.float32),
    )(x, batch_ids, node_time, seed_time,
      W_feat, b_feat, W_node, b_node, ln_g, ln_b, W_time, b_time)


def _layer_body(h_ref, p0_ref, p1_ref, ws_ref, wn_ref, b_ref, o_ref):
    agg = p0_ref[0] + p1_ref[0]
    o = (jnp.dot(h_ref[...], ws_ref[...], preferred_element_type=jnp.float32)
         + jnp.dot(agg, wn_ref[...], preferred_element_type=jnp.float32)
         + b_ref[...])
    o_ref[...] = jnp.maximum(o, 0.0)


def _tc_layer1(h, p, W_self, W_neigh, b):
    grid = N // _RBLK
    blk = pl.BlockSpec((_RBLK, C), lambda i: (i, 0))
    p0s = pl.BlockSpec((1, _RBLK, C), lambda i: (0, i, 0))
    p1s = pl.BlockSpec((1, _RBLK, C), lambda i: (1, i, 0))
    w2 = pl.BlockSpec((C, C), lambda i: (0, 0))
    w1 = pl.BlockSpec((C,), lambda i: (0,))
    return pl.pallas_call(
        _layer_body,
        grid=(grid,),
        in_specs=[blk, p0s, p1s, w2, w2, w1],
        out_specs=blk,
        out_shape=jax.ShapeDtypeStruct((N, C), jnp.float32),
    )(h, p, p, W_self, W_neigh, b)


def _head_body(h_ref, q0_ref, q1_ref, ws_ref, wn_ref, b_ref,
               wh_ref, bh_ref, o_ref):
    agg = q0_ref[0] + q1_ref[0]
    t = (jnp.dot(h_ref[...], ws_ref[...], preferred_element_type=jnp.float32)
         + jnp.dot(agg, wn_ref[...], preferred_element_type=jnp.float32)
         + b_ref[...])
    t = jnp.maximum(t, 0.0)
    o_ref[...] = jnp.dot(t, wh_ref[...],
                         preferred_element_type=jnp.float32) + bh_ref[...]


def _tc_head(h1, q, W_self, W_neigh, b, W_head, b_head):
    return pl.pallas_call(
        _head_body,
        grid=(1,),
        in_specs=[
            pl.BlockSpec((SEEDS, C), lambda i: (0, 0)),
            pl.BlockSpec((1, SEEDS, C), lambda i: (0, 0, 0)),
            pl.BlockSpec((1, SEEDS, C), lambda i: (1, 0, 0)),
            pl.BlockSpec((C, C), lambda i: (0, 0)),
            pl.BlockSpec((C, C), lambda i: (0, 0)),
            pl.BlockSpec((C,), lambda i: (0,)),
            pl.BlockSpec((C, OUT), lambda i: (0, 0)),
            pl.BlockSpec((OUT,), lambda i: (0,)),
        ],
        out_specs=pl.BlockSpec((SEEDS, OUT), lambda i: (0, 0)),
        out_shape=jax.ShapeDtypeStruct((SEEDS, OUT), jnp.float32),
    )(h1, q, q, W_self, W_neigh, b, W_head, b_head)


def kernel(x, edge_index, node_time, seed_time, batch_ids,
           W_feat, b_feat, W_node, b_node, ln_g, ln_b,
           W_time, b_time,
           W_self1, W_neigh1, b1, W_self2, W_neigh2, b2,
           W_head, b_head):
    h = _tc_encoder(x, batch_ids.reshape(N, 1), node_time.reshape(N, 1),
                    seed_time.reshape(1, SEEDS),
                    W_feat, b_feat, W_node, b_node, ln_g, ln_b,
                    W_time, b_time)

    zeros_hbm = jnp.zeros((ZSTRIPE, C), jnp.float32)
    src = edge_index[0]
    dst = edge_index[1]
    p, cpk, cnt = _sc_scatter(h, src, dst, zeros_hbm)
    h1 = _tc_layer1(h, p, W_self1, W_neigh1, b1)

    q = _sc_scatter_seeds(h1, cpk, cnt, zeros_hbm)
    return _tc_head(h1, q, W_self2, W_neigh2, b2, W_head, b_head)
